# Initial kernel scaffold; baseline (speedup 1.0000x reference)
import jax
import jax.numpy as jnp
from jax.experimental import pallas as pl


def kernel(query, memory_slots, top_k):
    raise NotImplementedError("write your pallas kernel here")


# speedup vs baseline: 3.2084x; 3.2084x over previous
import functools

import jax
import jax.numpy as jnp
from jax import lax
from jax.experimental import pallas as pl
from jax.experimental.pallas import tpu as pltpu
from jax.experimental.pallas import tpu_sc as plsc

Q = 1024
K = 8192
D = 2048
TOPK = 16

_BQ = 256
_BK = 512


def _sims_body(q_ref, m_ref, out_ref):
    q = q_ref[...]
    m = m_ref[...]
    dots = lax.dot_general(q, m, (((1,), (1,)), ((), ())),
                           preferred_element_type=jnp.float32)
    qn = jnp.sqrt(jnp.sum(q * q, axis=1))
    mn = jnp.sqrt(jnp.sum(m * m, axis=1))
    denom = jnp.maximum(qn[:, None] * mn[None, :], jnp.float32(1e-8))
    out_ref[...] = dots / denom


def _sims(query, memory_slots):
    grid = (Q // _BQ, K // _BK)
    return pl.pallas_call(
        _sims_body,
        grid=grid,
        in_specs=[
            pl.BlockSpec((_BQ, D), lambda i, j: (i, 0)),
            pl.BlockSpec((_BK, D), lambda i, j: (j, 0)),
        ],
        out_specs=pl.BlockSpec((_BQ, _BK), lambda i, j: (i, j)),
        out_shape=jax.ShapeDtypeStruct((Q, K), jnp.float32),
    )(query, memory_slots)


_TQ = 128


def _topk_body(s_ref, idx_ref):
    s = s_ref[...]
    gidx = lax.broadcasted_iota(jnp.int32, (_TQ, K), 1)
    neg_inf = jnp.float32(-jnp.inf)
    big = jnp.int32(2**31 - 1)
    cols = []
    for _ in range(TOPK):
        m = jnp.max(s, axis=1, keepdims=True)
        is_max = s == m
        g = jnp.min(jnp.where(is_max, gidx, big), axis=1, keepdims=True)
        cols.append(g)
        s = jnp.where(gidx == g, neg_inf, s)
    idx_ref[...] = jnp.concatenate(cols, axis=1)


def _topk(sims):
    return pl.pallas_call(
        _topk_body,
        grid=(Q // _TQ,),
        in_specs=[pl.BlockSpec((_TQ, K), lambda i: (i, 0))],
        out_specs=pl.BlockSpec((_TQ, TOPK), lambda i: (i, 0)),
        out_shape=jax.ShapeDtypeStruct((Q, TOPK), jnp.int32),
    )(sims)


_CH = 32


def _sc_gather(table, idx_flat):
    B = idx_flat.shape[0]
    NC, NS = 2, 16
    NW = NC * NS
    b_per_w = B // NW
    n_ch = b_per_w // _CH
    mesh = plsc.VectorSubcoreMesh(core_axis_name="c", subcore_axis_name="s")

    @functools.partial(
        pl.kernel,
        mesh=mesh,
        out_type=jax.ShapeDtypeStruct((B, D), jnp.float32),
        scratch_types=[
            pltpu.VMEM((_CH,), jnp.int32),
            pltpu.VMEM((_CH, D), jnp.float32),
            pltpu.SemaphoreType.DMA,
        ],
    )
    def k(table_hbm, idx_hbm, out_hbm, idx_v, rows_v, sem):
        wid = lax.axis_index("s") * NC + lax.axis_index("c")
        base = wid * b_per_w

        def body(c, carry):
            off = base + c * _CH
            pltpu.sync_copy(idx_hbm.at[pl.ds(off, _CH)], idx_v)
            pltpu.async_copy(table_hbm.at[idx_v], rows_v, sem).wait()
            pltpu.sync_copy(rows_v, out_hbm.at[pl.ds(off, _CH)])
            return carry

        lax.fori_loop(0, n_ch, body, 0)

    return k(table, idx_flat)


def kernel(query, memory_slots, top_k):
    sims = _sims(query, memory_slots)
    idx = _topk(sims)
    rows = _sc_gather(memory_slots, idx.reshape(-1))
    return rows.reshape(Q, TOPK, D)


# speedup vs baseline: 3.8626x; 1.2039x over previous
import functools

import jax
import jax.numpy as jnp
from jax import lax
from jax.experimental import pallas as pl
from jax.experimental.pallas import tpu as pltpu
from jax.experimental.pallas import tpu_sc as plsc

Q = 1024
K = 8192
D = 2048
TOPK = 16

_BQ = 256
_BK = 512


def _sims_body(q_ref, m_ref, out_ref):
    q = q_ref[...]
    m = m_ref[...]
    dots = lax.dot_general(q, m, (((1,), (1,)), ((), ())),
                           preferred_element_type=jnp.float32)
    qn = jnp.sqrt(jnp.sum(q * q, axis=1))
    mn = jnp.sqrt(jnp.sum(m * m, axis=1))
    denom = jnp.maximum(qn[:, None] * mn[None, :], jnp.float32(1e-8))
    out_ref[...] = dots / denom


def _sims(query, memory_slots):
    grid = (Q // _BQ, K // _BK)
    return pl.pallas_call(
        _sims_body,
        grid=grid,
        in_specs=[
            pl.BlockSpec((_BQ, D), lambda i, j: (i, 0)),
            pl.BlockSpec((_BK, D), lambda i, j: (j, 0)),
        ],
        out_specs=pl.BlockSpec((_BQ, _BK), lambda i, j: (i, j)),
        out_shape=jax.ShapeDtypeStruct((Q, K), jnp.float32),
    )(query, memory_slots)


_TQ = 128


_NP = 8
_GW = K // _NP


def _iter_topk(vals, gids, width):
    neg_inf = jnp.float32(-jnp.inf)
    big = jnp.int32(2**31 - 1)
    cols = []
    w16 = None
    for _ in range(TOPK):
        m = jnp.max(vals, axis=1, keepdims=True)
        g = jnp.min(jnp.where(vals == m, gids, big), axis=1, keepdims=True)
        cols.append(g)
        w16 = m
        vals = jnp.where(gids == g, neg_inf, vals)
    return jnp.concatenate(cols, axis=1), w16


def _topk_body(s_ref, idx_ref):
    neg_inf = jnp.float32(-jnp.inf)
    lane = lax.broadcasted_iota(jnp.int32, (_TQ, _GW), 1)
    v1 = s_ref[:, 0:_GW]
    g1 = lane
    v2 = jnp.full((_TQ, _GW), neg_inf)
    v3 = v2
    g2 = jnp.zeros((_TQ, _GW), jnp.int32)
    g3 = g2
    for p in range(1, _NP):
        x = s_ref[:, p * _GW:(p + 1) * _GW]
        gx = lane + jnp.int32(p * _GW)
        b1 = x > v1
        b2 = x > v2
        b3 = x > v3
        v3 = jnp.where(b2, v2, jnp.where(b3, x, v3))
        g3 = jnp.where(b2, g2, jnp.where(b3, gx, g3))
        v2 = jnp.where(b1, v1, jnp.where(b2, x, v2))
        g2 = jnp.where(b1, g1, jnp.where(b2, gx, g2))
        v1 = jnp.where(b1, x, v1)
        g1 = jnp.where(b1, gx, g1)
    cand_v = jnp.concatenate([v1, v2, v3], axis=1)
    cand_g = jnp.concatenate([g1, g2, g3], axis=1)
    idx, w16 = _iter_topk(cand_v, cand_g, 3 * _GW)
    idx_ref[...] = idx
    fb = jnp.any(jnp.max(v3, axis=1, keepdims=True) >= w16)

    @pl.when(fb)
    def _fallback():
        gidx = lax.broadcasted_iota(jnp.int32, (_TQ, K), 1)
        idx_full, _ = _iter_topk(s_ref[...], gidx, K)
        idx_ref[...] = idx_full


def _topk(sims):
    return pl.pallas_call(
        _topk_body,
        grid=(Q // _TQ,),
        in_specs=[pl.BlockSpec((_TQ, K), lambda i: (i, 0))],
        out_specs=pl.BlockSpec((_TQ, TOPK), lambda i: (i, 0)),
        out_shape=jax.ShapeDtypeStruct((Q, TOPK), jnp.int32),
    )(sims)


_CH = 32


def _sc_gather(table, idx_flat):
    B = idx_flat.shape[0]
    NC, NS = 2, 16
    NW = NC * NS
    b_per_w = B // NW
    n_ch = b_per_w // _CH
    mesh = plsc.VectorSubcoreMesh(core_axis_name="c", subcore_axis_name="s")

    @functools.partial(
        pl.kernel,
        mesh=mesh,
        out_type=jax.ShapeDtypeStruct((B, D), jnp.float32),
        scratch_types=[
            pltpu.VMEM((_CH,), jnp.int32),
            pltpu.VMEM((_CH, D), jnp.float32),
            pltpu.SemaphoreType.DMA,
        ],
    )
    def k(table_hbm, idx_hbm, out_hbm, idx_v, rows_v, sem):
        wid = lax.axis_index("s") * NC + lax.axis_index("c")
        base = wid * b_per_w

        def body(c, carry):
            off = base + c * _CH
            pltpu.sync_copy(idx_hbm.at[pl.ds(off, _CH)], idx_v)
            pltpu.async_copy(table_hbm.at[idx_v], rows_v, sem).wait()
            pltpu.sync_copy(rows_v, out_hbm.at[pl.ds(off, _CH)])
            return carry

        lax.fori_loop(0, n_ch, body, 0)

    return k(table, idx_flat)


def kernel(query, memory_slots, top_k):
    sims = _sims(query, memory_slots)
    idx = _topk(sims)
    rows = _sc_gather(memory_slots, idx.reshape(-1))
    return rows.reshape(Q, TOPK, D)


# speedup vs baseline: 3.9504x; 1.0227x over previous
import functools

import jax
import jax.numpy as jnp
from jax import lax
from jax.experimental import pallas as pl
from jax.experimental.pallas import tpu as pltpu
from jax.experimental.pallas import tpu_sc as plsc

Q = 1024
K = 8192
D = 2048
TOPK = 16

_BQ = 256
_BK = 512


def _sims_body(q_ref, m_ref, out_ref):
    q = q_ref[...]
    m = m_ref[...]
    dots = lax.dot_general(q, m, (((1,), (1,)), ((), ())),
                           preferred_element_type=jnp.float32)
    qn = jnp.sqrt(jnp.sum(q * q, axis=1))
    mn = jnp.sqrt(jnp.sum(m * m, axis=1))
    denom = jnp.maximum(qn[:, None] * mn[None, :], jnp.float32(1e-8))
    out_ref[...] = dots / denom


def _sims(query, memory_slots):
    grid = (Q // _BQ, K // _BK)
    return pl.pallas_call(
        _sims_body,
        grid=grid,
        in_specs=[
            pl.BlockSpec((_BQ, D), lambda i, j: (i, 0)),
            pl.BlockSpec((_BK, D), lambda i, j: (j, 0)),
        ],
        out_specs=pl.BlockSpec((_BQ, _BK), lambda i, j: (i, j)),
        out_shape=jax.ShapeDtypeStruct((Q, K), jnp.float32),
    )(query, memory_slots)


_TQ = 128


_NP = 8
_GW = K // _NP


def _iter_topk(vals, gids, width):
    neg_inf = jnp.float32(-jnp.inf)
    big = jnp.int32(2**31 - 1)
    cols = []
    w16 = None
    for _ in range(TOPK):
        m = jnp.max(vals, axis=1, keepdims=True)
        g = jnp.min(jnp.where(vals == m, gids, big), axis=1, keepdims=True)
        cols.append(g)
        w16 = m
        vals = jnp.where(gids == g, neg_inf, vals)
    return jnp.concatenate(cols, axis=1), w16


def _topk_body(s_ref, idx_ref):
    neg_inf = jnp.float32(-jnp.inf)
    lane = lax.broadcasted_iota(jnp.int32, (_TQ, _GW), 1)
    v1 = s_ref[:, 0:_GW]
    g1 = lane
    v2 = jnp.full((_TQ, _GW), neg_inf)
    v3 = v2
    g2 = jnp.zeros((_TQ, _GW), jnp.int32)
    g3 = g2
    for p in range(1, _NP):
        x = s_ref[:, p * _GW:(p + 1) * _GW]
        gx = lane + jnp.int32(p * _GW)
        b1 = x > v1
        b2 = x > v2
        b3 = x > v3
        v3 = jnp.where(b2, v2, jnp.where(b3, x, v3))
        g3 = jnp.where(b2, g2, jnp.where(b3, gx, g3))
        v2 = jnp.where(b1, v1, jnp.where(b2, x, v2))
        g2 = jnp.where(b1, g1, jnp.where(b2, gx, g2))
        v1 = jnp.where(b1, x, v1)
        g1 = jnp.where(b1, gx, g1)
    cand_v = jnp.concatenate([v1, v2, v3], axis=1)
    cand_g = jnp.concatenate([g1, g2, g3], axis=1)
    idx, w16 = _iter_topk(cand_v, cand_g, 3 * _GW)
    idx_ref[...] = idx
    fb = jnp.any(jnp.max(v3, axis=1, keepdims=True) >= w16)

    @pl.when(fb)
    def _fallback():
        gidx = lax.broadcasted_iota(jnp.int32, (_TQ, K), 1)
        idx_full, _ = _iter_topk(s_ref[...], gidx, K)
        idx_ref[...] = idx_full


def _topk(sims):
    return pl.pallas_call(
        _topk_body,
        grid=(Q // _TQ,),
        in_specs=[pl.BlockSpec((_TQ, K), lambda i: (i, 0))],
        out_specs=pl.BlockSpec((_TQ, TOPK), lambda i: (i, 0)),
        out_shape=jax.ShapeDtypeStruct((Q, TOPK), jnp.int32),
    )(sims)


_CH = 16


def _sc_gather(table, idx_flat):
    B = idx_flat.shape[0]
    NC, NS = 2, 16
    NW = NC * NS
    b_per_w = B // NW
    n_ch = b_per_w // _CH
    mesh = plsc.VectorSubcoreMesh(core_axis_name="c", subcore_axis_name="s")

    @functools.partial(
        pl.kernel,
        mesh=mesh,
        out_type=jax.ShapeDtypeStruct((B, D), jnp.float32),
        scratch_types=[
            pltpu.VMEM((b_per_w,), jnp.int32),
            pltpu.VMEM((_CH, D), jnp.float32),
            pltpu.VMEM((_CH, D), jnp.float32),
            pltpu.SemaphoreType.DMA,
            pltpu.SemaphoreType.DMA,
            pltpu.SemaphoreType.DMA,
            pltpu.SemaphoreType.DMA,
        ],
    )
    def k(table_hbm, idx_hbm, out_hbm, idx_v, buf0, buf1, g0, g1, s0, s1):
        wid = lax.axis_index("s") * NC + lax.axis_index("c")
        base = wid * b_per_w
        pltpu.sync_copy(idx_hbm.at[pl.ds(base, b_per_w)], idx_v)
        bufs = (buf0, buf1)
        gsems = (g0, g1)
        ssems = (s0, s1)

        def gather(c):
            return pltpu.async_copy(
                table_hbm.at[idx_v.at[pl.ds(c * _CH, _CH)]],
                bufs[c % 2], gsems[c % 2])

        def scatter(c):
            return pltpu.async_copy(
                bufs[c % 2], out_hbm.at[pl.ds(base + c * _CH, _CH)],
                ssems[c % 2])

        pend_g = gather(0)
        pend_s = {}
        for c in range(n_ch):
            pend_g.wait()
            if c >= 1:
                pend_s[c - 1].wait()
            if c + 1 < n_ch:
                pend_g = gather(c + 1)
            pend_s[c] = scatter(c)
        pend_s[n_ch - 1].wait()

    return k(table, idx_flat)


def kernel(query, memory_slots, top_k):
    sims = _sims(query, memory_slots)
    idx = _topk(sims)
    rows = _sc_gather(memory_slots, idx.reshape(-1))
    return rows.reshape(Q, TOPK, D)
